# baseline (speedup 1.0000x reference)
import functools

import jax
import jax.numpy as jnp
from jax import lax
from jax.experimental import pallas as pl
from jax.experimental.pallas import tpu as pltpu
from jax.experimental.pallas import tpu_sc as plsc

B = 16384
V = 10
D = 512
NC = 2
NS = 16
NW = NC * NS
BPW = B // NW
CH = 64
NCHUNK = BPW // CH

_mesh = plsc.VectorSubcoreMesh(
    core_axis_name="c", subcore_axis_name="s", num_cores=NC, num_subcores=NS
)


@functools.partial(
    pl.kernel,
    out_type=jax.ShapeDtypeStruct((B, D), jnp.int32),
    mesh=_mesh,
    scratch_types=[
        pltpu.VMEM((NCHUNK, CH), jnp.int32),
        pltpu.VMEM((2, CH, D), jnp.int32),
        pltpu.SemaphoreType.DMA,
        pltpu.SemaphoreType.DMA,
        pltpu.SemaphoreType.DMA,
        pltpu.SemaphoreType.DMA,
    ],
)
def _gather_sc(tbl_hbm, idx_hbm, out_hbm, idx_v, rows_v, g0, g1, s0, s1):
    wid = lax.axis_index("s") * NC + lax.axis_index("c")
    base = wid * BPW
    gsem = (g0, g1)
    ssem = (s0, s1)

    for c in range(NCHUNK):
        pltpu.sync_copy(idx_hbm.at[pl.ds(base + c * CH, CH)], idx_v.at[c])

    def fire_gather(c):
        return pltpu.async_copy(
            tbl_hbm.at[idx_v.at[c]], rows_v.at[c % 2], gsem[c % 2]
        )

    def fire_store(c):
        return pltpu.async_copy(
            rows_v.at[c % 2], out_hbm.at[pl.ds(base + c * CH, CH)], ssem[c % 2]
        )

    gats = {}
    for c in range(NCHUNK):
        if c >= 2:
            gats[c - 2].wait()
        gats[c] = fire_gather(c)
    gats[NCHUNK - 2].wait()
    gats[NCHUNK - 1].wait()
    st = fire_store(0)
    st.wait()


def kernel(ind, mem):
    tbl = mem.astype(jnp.int32)
    idx = ind.astype(jnp.int32)
    return _gather_sc(tbl, idx)


# speedup vs baseline: 1.6068x; 1.6068x over previous
import functools

import jax
import jax.numpy as jnp
from jax import lax
from jax.experimental import pallas as pl
from jax.experimental.pallas import tpu as pltpu
from jax.experimental.pallas import tpu_sc as plsc

B = 16384
V = 10
D = 512
NC = 2
NS = 16
NW = NC * NS
BPW = B // NW
CH = 64
NCHUNK = BPW // CH
L = 16

_mesh = plsc.VectorSubcoreMesh(
    core_axis_name="c", subcore_axis_name="s", num_cores=NC, num_subcores=NS
)


@functools.partial(
    pl.kernel,
    out_type=jax.ShapeDtypeStruct((B, D), jnp.int32),
    mesh=_mesh,
    scratch_types=[
        pltpu.VMEM((NCHUNK, CH), jnp.int32),
        pltpu.VMEM((2, CH, D), jnp.int32),
        pltpu.SemaphoreType.DMA,
        pltpu.SemaphoreType.DMA,
        pltpu.SemaphoreType.DMA,
        pltpu.SemaphoreType.DMA,
    ],
)
def _gather_sc(tbl_hbm, idx_hbm, out_hbm, idx_v, rows_v, g0, g1, s0, s1):
    wid = lax.axis_index("s") * NC + lax.axis_index("c")
    base = wid * BPW
    gsem = (g0, g1)
    ssem = (s0, s1)

    for c in range(NCHUNK):
        pltpu.sync_copy(idx_hbm.at[pl.ds(base + c * CH, CH)], idx_v.at[c])

    off = wid * V
    for c in range(NCHUNK):
        for j in range(CH // L):
            sl = pl.ds(j * L, L)
            idx_v[c, sl] = idx_v[c, sl] + off

    def fire_gather(c):
        return pltpu.async_copy(
            tbl_hbm.at[idx_v.at[c]], rows_v.at[c % 2], gsem[c % 2]
        )

    def fire_store(c):
        return pltpu.async_copy(
            rows_v.at[c % 2], out_hbm.at[pl.ds(base + c * CH, CH)], ssem[c % 2]
        )

    gat = fire_gather(0)
    stores = {}
    for c in range(NCHUNK):
        if c + 1 < NCHUNK:
            if c - 1 >= 0:
                stores[c - 1].wait()
            nxt = fire_gather(c + 1)
        gat.wait()
        stores[c] = fire_store(c)
        if c + 1 < NCHUNK:
            gat = nxt
    stores[NCHUNK - 2].wait()
    stores[NCHUNK - 1].wait()


def kernel(ind, mem):
    tbl = jnp.broadcast_to(mem.astype(jnp.int32), (NW, V, D)).reshape(NW * V, D)
    idx = ind.astype(jnp.int32)
    return _gather_sc(tbl, idx)
